# baseline (speedup 1.0000x reference)
import jax
import jax.numpy as jnp
from jax.experimental import pallas as pl

_N, _C = 100000, 240
_RB = 1000


def _permute_body(x_ref, o_ref):
    x = x_ref[...]
    o_ref[...] = jnp.concatenate(
        [x[:, 96:160], x[:, 0:96], x[:, 160:240]], axis=-1
    )


def kernel(x):
    return pl.pallas_call(
        _permute_body,
        grid=(_N // _RB,),
        in_specs=[pl.BlockSpec((_RB, _C), lambda i: (i, 0))],
        out_specs=pl.BlockSpec((_RB, _C), lambda i: (i, 0)),
        out_shape=jax.ShapeDtypeStruct((_N, _C), x.dtype),
    )(x)
